# baseline (speedup 1.0000x reference)
import functools

import jax
import jax.numpy as jnp
from jax import lax
from jax.experimental import pallas as pl
from jax.experimental.pallas import tpu as pltpu
from jax.experimental.pallas import tpu_sc as plsc

L = 16
NW = 32
TOK_W = 512
NCHUNK = 4

MULT_CUR = 36313
MULT_PREV = 27191


def _hash16(cur, prev, mod, inv):
    v = (cur * MULT_CUR) ^ (prev * MULT_PREV)
    q = (v.astype(jnp.float32) * inv).astype(jnp.int32)
    r = v - q * mod
    r = jnp.where(r < 0, r + mod, r)
    r = jnp.where(r >= mod, r - mod, r)
    return r


def _body(n_tok_row, mod, x_hbm, scale_hbm, embed_hbm, out_hbm,
          xbuf, rows, sbuf, *sems):
    wid = lax.axis_index("s") * 2 + lax.axis_index("c")
    p = pl.multiple_of(wid * TOK_W, TOK_W)
    row_workers = n_tok_row // TOK_W
    row_flag = jnp.minimum(wid & (row_workers - 1), 1)

    pltpu.sync_copy(x_hbm.at[pl.ds(p, TOK_W)], xbuf.at[pl.ds(8, TOK_W)])
    pb = pl.multiple_of(jnp.maximum(p - 8, 0), 8)
    pltpu.sync_copy(x_hbm.at[pl.ds(pb, 8)], xbuf.at[pl.ds(0, 8)])
    pltpu.sync_copy(scale_hbm, sbuf)

    iota = lax.broadcasted_iota(jnp.int32, (L,), 0)
    inv = jnp.float32(1.0) / jnp.float32(mod)
    sv = sbuf[...]
    d = rows.shape[1]
    n_grp = TOK_W // L
    grp_per_chunk = n_grp // NCHUNK

    def fire16(k, hv, csem):
        for j in range(L):
            h = hv[j]
            pltpu.async_copy(embed_hbm.at[h >> 3, h & 7], rows.at[k * L + j],
                             csem)

    def hash_grp(k, first):
        cur = xbuf[pl.ds(8 + k * L, L)]
        prev = xbuf[pl.ds(7 + k * L, L)]
        if first:
            prev = prev * jnp.where(iota == 0, row_flag, 1)
        return _hash16(cur, prev, mod, inv)

    for c in range(NCHUNK):
        csem = sems[c]

        def fire_body(k, carry, _csem=csem):
            fire16(k, hash_grp(k, first=False), _csem)
            return carry

        g0 = c * grp_per_chunk
        if c == 0:
            fire16(0, hash_grp(0, first=True), csem)
            g0 = 1
        lax.fori_loop(g0, (c + 1) * grp_per_chunk, fire_body, 0)

    tok_c = TOK_W // NCHUNK
    for c in range(NCHUNK):
        t0 = c * tok_c
        chunk = rows.at[pl.ds(t0, tok_c)]
        pltpu.make_async_copy(out_hbm.at[pl.ds(0, tok_c)],
                              chunk, sems[c]).wait()

        def mul_body(g, carry, _t0=t0):
            r0 = _t0 + g * L
            for rr in range(L):
                for c0 in range(d // L):
                    sl = pl.ds(c0 * L, L)
                    rows[r0 + rr, sl] = rows[r0 + rr, sl] * sv
            return carry

        lax.fori_loop(0, grp_per_chunk, mul_body, 0)
        pltpu.sync_copy(chunk, out_hbm.at[pl.ds(p + t0, tok_c)])


def kernel(x, embed, scale):
    b, s = x.shape
    v, d = embed.shape
    xf = x.reshape(-1)
    scale16 = jnp.full((L,), scale, jnp.float32)
    mesh = plsc.VectorSubcoreMesh(core_axis_name="c", subcore_axis_name="s")
    run = pl.kernel(
        functools.partial(_body, s, v - 1),
        mesh=mesh,
        out_type=jax.ShapeDtypeStruct((b * s, d), jnp.float32),
        scratch_types=[
            pltpu.VMEM((TOK_W + 8,), jnp.int32),
            pltpu.VMEM((TOK_W, d), jnp.float32),
            pltpu.VMEM((L,), jnp.float32),
        ] + [pltpu.SemaphoreType.DMA] * NCHUNK,
    )
    out = run(xf, scale16, embed.reshape(v // 8, 8, d))
    return out.reshape(b, s, d)


# speedup vs baseline: 1.0039x; 1.0039x over previous
import functools

import jax
import jax.numpy as jnp
from jax import lax
from jax.experimental import pallas as pl
from jax.experimental.pallas import tpu as pltpu
from jax.experimental.pallas import tpu_sc as plsc

L = 16
NW = 32
TOK_W = 512
NCHUNK = 2

MULT_CUR = 36313
MULT_PREV = 27191


def _hash16(cur, prev, mod, inv):
    v = (cur * MULT_CUR) ^ (prev * MULT_PREV)
    q = (v.astype(jnp.float32) * inv).astype(jnp.int32)
    r = v - q * mod
    r = jnp.where(r < 0, r + mod, r)
    r = jnp.where(r >= mod, r - mod, r)
    return r


def _body(n_tok_row, mod, x_hbm, scale_hbm, embed_hbm, out_hbm,
          xbuf, rows, sbuf, *sems):
    wid = lax.axis_index("s") * 2 + lax.axis_index("c")
    p = pl.multiple_of(wid * TOK_W, TOK_W)
    row_workers = n_tok_row // TOK_W
    row_flag = jnp.minimum(wid & (row_workers - 1), 1)

    pltpu.sync_copy(x_hbm.at[pl.ds(p, TOK_W)], xbuf.at[pl.ds(8, TOK_W)])
    pb = pl.multiple_of(jnp.maximum(p - 8, 0), 8)
    pltpu.sync_copy(x_hbm.at[pl.ds(pb, 8)], xbuf.at[pl.ds(0, 8)])
    pltpu.sync_copy(scale_hbm, sbuf)

    iota = lax.broadcasted_iota(jnp.int32, (L,), 0)
    inv = jnp.float32(1.0) / jnp.float32(mod)
    sv = sbuf[...]
    d = rows.shape[1]
    n_grp = TOK_W // L
    grp_per_chunk = n_grp // NCHUNK

    def fire16(k, hv, csem):
        for j in range(L):
            h = hv[j]
            pltpu.async_copy(embed_hbm.at[h >> 3, h & 7], rows.at[k * L + j],
                             csem)

    def hash_grp(k, first):
        cur = xbuf[pl.ds(8 + k * L, L)]
        prev = xbuf[pl.ds(7 + k * L, L)]
        if first:
            prev = prev * jnp.where(iota == 0, row_flag, 1)
        return _hash16(cur, prev, mod, inv)

    sem = sems[0]
    fire16(0, hash_grp(0, first=True), sem)

    def fire_body(k, carry):
        fire16(k, hash_grp(k, first=False), sem)
        return carry

    lax.fori_loop(1, n_grp, fire_body, 0)

    pltpu.make_async_copy(out_hbm.at[pl.ds(0, TOK_W)], rows, sem).wait()

    tok_c = TOK_W // NCHUNK
    for c in range(NCHUNK):
        t0 = c * tok_c

        def mul_body(g, carry, _t0=t0):
            r0 = _t0 + g * L
            for rr in range(L):
                for c0 in range(d // L):
                    sl = pl.ds(c0 * L, L)
                    rows[r0 + rr, sl] = rows[r0 + rr, sl] * sv
            return carry

        lax.fori_loop(0, grp_per_chunk, mul_body, 0)
        pltpu.sync_copy(rows.at[pl.ds(t0, tok_c)],
                        out_hbm.at[pl.ds(p + t0, tok_c)])


def kernel(x, embed, scale):
    b, s = x.shape
    v, d = embed.shape
    xf = x.reshape(-1)
    scale16 = jnp.full((L,), scale, jnp.float32)
    mesh = plsc.VectorSubcoreMesh(core_axis_name="c", subcore_axis_name="s")
    run = pl.kernel(
        functools.partial(_body, s, v - 1),
        mesh=mesh,
        out_type=jax.ShapeDtypeStruct((b * s, d), jnp.float32),
        scratch_types=[
            pltpu.VMEM((TOK_W + 8,), jnp.int32),
            pltpu.VMEM((TOK_W, d), jnp.float32),
            pltpu.VMEM((L,), jnp.float32),
        ] + [pltpu.SemaphoreType.DMA],
    )
    out = run(xf, scale16, embed.reshape(v // 8, 8, d))
    return out.reshape(b, s, d)


# speedup vs baseline: 1.0055x; 1.0017x over previous
import functools

import jax
import jax.numpy as jnp
from jax import lax
from jax.experimental import pallas as pl
from jax.experimental.pallas import tpu as pltpu
from jax.experimental.pallas import tpu_sc as plsc

L = 16
NW = 32
TOK_W = 512
NCHUNK = 1

MULT_CUR = 36313
MULT_PREV = 27191


def _hash16(cur, prev, mod, inv):
    v = (cur * MULT_CUR) ^ (prev * MULT_PREV)
    q = (v.astype(jnp.float32) * inv).astype(jnp.int32)
    r = v - q * mod
    r = jnp.where(r < 0, r + mod, r)
    r = jnp.where(r >= mod, r - mod, r)
    return r


def _body(n_tok_row, mod, x_hbm, scale_hbm, embed_hbm, out_hbm,
          xbuf, rows, sbuf, *sems):
    wid = lax.axis_index("s") * 2 + lax.axis_index("c")
    p = pl.multiple_of(wid * TOK_W, TOK_W)
    row_workers = n_tok_row // TOK_W
    row_flag = jnp.minimum(wid & (row_workers - 1), 1)

    pltpu.sync_copy(x_hbm.at[pl.ds(p, TOK_W)], xbuf.at[pl.ds(8, TOK_W)])
    pb = pl.multiple_of(jnp.maximum(p - 8, 0), 8)
    pltpu.sync_copy(x_hbm.at[pl.ds(pb, 8)], xbuf.at[pl.ds(0, 8)])
    pltpu.sync_copy(scale_hbm, sbuf)

    iota = lax.broadcasted_iota(jnp.int32, (L,), 0)
    inv = jnp.float32(1.0) / jnp.float32(mod)
    sv = sbuf[...]
    d = rows.shape[1]
    n_grp = TOK_W // L
    grp_per_chunk = n_grp // NCHUNK

    def fire16(k, hv, csem):
        for j in range(L):
            h = hv[j]
            pltpu.async_copy(embed_hbm.at[h >> 3, h & 7], rows.at[k * L + j],
                             csem)

    def hash_grp(k, first):
        cur = xbuf[pl.ds(8 + k * L, L)]
        prev = xbuf[pl.ds(7 + k * L, L)]
        if first:
            prev = prev * jnp.where(iota == 0, row_flag, 1)
        return _hash16(cur, prev, mod, inv)

    sem = sems[0]
    fire16(0, hash_grp(0, first=True), sem)

    def fire_body(k, carry):
        fire16(k, hash_grp(k, first=False), sem)
        return carry

    lax.fori_loop(1, n_grp, fire_body, 0)

    pltpu.make_async_copy(out_hbm.at[pl.ds(0, TOK_W)], rows, sem).wait()

    tok_c = TOK_W // NCHUNK
    for c in range(NCHUNK):
        t0 = c * tok_c

        def mul_body(g, carry, _t0=t0):
            r0 = _t0 + g * L
            for rr in range(L):
                for c0 in range(d // L):
                    sl = pl.ds(c0 * L, L)
                    rows[r0 + rr, sl] = rows[r0 + rr, sl] * sv
            return carry

        lax.fori_loop(0, grp_per_chunk, mul_body, 0)
        pltpu.sync_copy(rows.at[pl.ds(t0, tok_c)],
                        out_hbm.at[pl.ds(p + t0, tok_c)])


def kernel(x, embed, scale):
    b, s = x.shape
    v, d = embed.shape
    xf = x.reshape(-1)
    scale16 = jnp.full((L,), scale, jnp.float32)
    mesh = plsc.VectorSubcoreMesh(core_axis_name="c", subcore_axis_name="s")
    run = pl.kernel(
        functools.partial(_body, s, v - 1),
        mesh=mesh,
        out_type=jax.ShapeDtypeStruct((b * s, d), jnp.float32),
        scratch_types=[
            pltpu.VMEM((TOK_W + 8,), jnp.int32),
            pltpu.VMEM((TOK_W, d), jnp.float32),
            pltpu.VMEM((L,), jnp.float32),
        ] + [pltpu.SemaphoreType.DMA],
    )
    out = run(xf, scale16, embed.reshape(v // 8, 8, d))
    return out.reshape(b, s, d)
